# baseline (speedup 1.0000x reference)
import functools

import jax
import jax.numpy as jnp
from jax import lax
from jax.experimental import pallas as pl
from jax.experimental.pallas import tpu as pltpu
from jax.experimental.pallas import tpu_sc as plsc


def kernel(inputs, memory):
    B, D = inputs.shape
    L, _ = memory.shape
    info = plsc.get_sparse_core_info()
    NC, NS = info.num_cores, info.num_subcores
    NW = NC * NS
    assert B % NW == 0
    b_per_w = B // NW

    mesh = plsc.VectorSubcoreMesh(core_axis_name="c", subcore_axis_name="s")

    @functools.partial(
        pl.kernel,
        mesh=mesh,
        out_type=jax.ShapeDtypeStruct((B, L, D), jnp.float32),
        scratch_types=[
            pltpu.VMEM((L, D), jnp.float32),
            pltpu.VMEM((b_per_w, D), jnp.float32),
            pltpu.SemaphoreType.DMA,
            pltpu.SemaphoreType.DMA,
        ],
        compiler_params=pltpu.CompilerParams(use_tc_tiling_on_sc=False),
    )
    def _sc_broadcast(inputs_hbm, memory_hbm, out_hbm, mem_v, in_v,
                      sem_big, sem_small):
        wid = lax.axis_index("s") * NC + lax.axis_index("c")
        base = wid * b_per_w
        mem_stage = pltpu.async_copy(memory_hbm, mem_v, sem_big)
        in_stage = pltpu.async_copy(
            inputs_hbm.at[pl.ds(base, b_per_w)], in_v, sem_small)
        in_stage.wait()
        copies = [pltpu.async_copy(
            in_v, out_hbm.at[pl.ds(base, b_per_w), 0], sem_small)]
        mem_stage.wait()
        for j in range(b_per_w):
            copies.append(pltpu.async_copy(
                mem_v.at[pl.ds(1, L - 1)],
                out_hbm.at[base + j, pl.ds(1, L - 1)],
                sem_big))
        for c in copies:
            c.wait()

    return _sc_broadcast(inputs, memory)


# speedup vs baseline: 1.0438x; 1.0438x over previous
import functools

import jax
import jax.numpy as jnp
from jax import lax
from jax.experimental import pallas as pl
from jax.experimental.pallas import tpu as pltpu
from jax.experimental.pallas import tpu_sc as plsc


def kernel(inputs, memory):
    B, D = inputs.shape
    L, _ = memory.shape
    info = plsc.get_sparse_core_info()
    NC, NS = info.num_cores, info.num_subcores
    NW = NC * NS
    assert B % NW == 0
    b_per_w = B // NW

    mesh = plsc.VectorSubcoreMesh(core_axis_name="c", subcore_axis_name="s")

    @functools.partial(
        pl.kernel,
        mesh=mesh,
        out_type=jax.ShapeDtypeStruct((B, L, D), jnp.float32),
        scratch_types=[
            pltpu.VMEM((L, D), jnp.float32),
            pltpu.VMEM((b_per_w, D), jnp.float32),
            pltpu.SemaphoreType.DMA,
            pltpu.SemaphoreType.DMA,
        ],
        compiler_params=pltpu.CompilerParams(use_tc_tiling_on_sc=False),
    )
    def _sc_broadcast(inputs_hbm, memory_hbm, out_hbm, mem_v, in_v,
                      sem_big, sem_small):
        wid = lax.axis_index("s") * NC + lax.axis_index("c")
        base = wid * b_per_w
        pltpu.sync_copy(memory_hbm, mem_v)
        pltpu.sync_copy(inputs_hbm.at[pl.ds(base, b_per_w)], in_v)
        copies = []
        for j in range(b_per_w):
            copies.append(pltpu.async_copy(
                mem_v.at[pl.ds(1, L - 1)],
                out_hbm.at[base + j, pl.ds(1, L - 1)],
                sem_big))
        copies.append(pltpu.async_copy(
            in_v, out_hbm.at[pl.ds(base, b_per_w), 0], sem_small))
        for c in copies:
            c.wait()

    return _sc_broadcast(inputs, memory)


# speedup vs baseline: 1.0522x; 1.0080x over previous
import functools

import jax
import jax.numpy as jnp
from jax import lax
from jax.experimental import pallas as pl
from jax.experimental.pallas import tpu as pltpu
from jax.experimental.pallas import tpu_sc as plsc


def kernel(inputs, memory):
    B, D = inputs.shape
    L, _ = memory.shape
    info = plsc.get_sparse_core_info()
    NC, NS = info.num_cores, info.num_subcores
    NW = NC * NS
    assert B % NW == 0
    b_per_w = B // NW

    mesh = plsc.VectorSubcoreMesh(core_axis_name="c", subcore_axis_name="s")

    @functools.partial(
        pl.kernel,
        mesh=mesh,
        out_type=jax.ShapeDtypeStruct((B, L, D), jnp.float32),
        scratch_types=[
            pltpu.VMEM((L, D), jnp.float32),
            pltpu.VMEM((b_per_w, D), jnp.float32),
            pltpu.SemaphoreType.DMA,
            pltpu.SemaphoreType.DMA,
        ],
        compiler_params=pltpu.CompilerParams(use_tc_tiling_on_sc=False),
    )
    def _sc_broadcast(inputs_hbm, memory_hbm, out_hbm, mem_v, in_v,
                      sem_big, sem_small):
        wid = lax.axis_index("s") * NC + lax.axis_index("c")
        base = wid * b_per_w
        pltpu.sync_copy(memory_hbm, mem_v)
        pltpu.sync_copy(inputs_hbm.at[pl.ds(base, b_per_w)], in_v)
        copies = [pltpu.async_copy(
            in_v, out_hbm.at[pl.ds(base, b_per_w), 0], sem_small)]
        for j in range(b_per_w):
            copies.append(pltpu.async_copy(
                mem_v.at[pl.ds(1, L - 1)],
                out_hbm.at[base + j, pl.ds(1, L - 1)],
                sem_big))
        for c in copies:
            c.wait()

    return _sc_broadcast(inputs, memory)
